# baseline (speedup 1.0000x reference)
import functools

import jax
import jax.numpy as jnp
from jax import lax
from jax.experimental import pallas as pl
from jax.experimental.pallas import tpu as pltpu
from jax.experimental.pallas import tpu_sc as plsc

BATCH = 16384
EMBED = 32
HIDDEN = 256

_NC, _NS = 2, 16
_NW = _NC * _NS
_B_PER_W = BATCH // _NW
_ICHUNK = 128
_NICHUNK = _B_PER_W // _ICHUNK


def _sc_gather(user_id, movie_id, utab128, mtab128):
    mesh = plsc.VectorSubcoreMesh(core_axis_name="c", subcore_axis_name="s")

    @functools.partial(
        pl.kernel,
        mesh=mesh,
        out_type=[
            pltpu.HBM((BATCH, 128), jnp.float32),
            pltpu.HBM((BATCH, 128), jnp.float32),
        ],
        scratch_types=[
            pltpu.VMEM((_B_PER_W,), jnp.int32),
            pltpu.VMEM((_B_PER_W,), jnp.int32),
            pltpu.VMEM((2, _ICHUNK, 128), jnp.float32),
            pltpu.VMEM((2, _ICHUNK, 128), jnp.float32),
            pltpu.SemaphoreType.DMA,
        ],
    )
    def k(uid_hbm, mid_hbm, utab_hbm, mtab_hbm, uout_hbm, mout_hbm,
          uidx_v, midx_v, ulines_v, mlines_v, sem):
        wid = lax.axis_index("s") * _NC + lax.axis_index("c")
        base = wid * _B_PER_W
        pltpu.sync_copy(uid_hbm.at[pl.ds(base, _B_PER_W)], uidx_v)
        pltpu.sync_copy(mid_hbm.at[pl.ds(base, _B_PER_W)], midx_v)

        def fire(c):
            sl = pl.ds(c * _ICHUNK, _ICHUNK)
            buf = c % 2
            ucp = pltpu.async_copy(
                utab_hbm.at[uidx_v.at[sl]], ulines_v.at[buf], sem)
            mcp = pltpu.async_copy(
                mtab_hbm.at[midx_v.at[sl]], mlines_v.at[buf], sem)
            return ucp, mcp

        cps = fire(0)
        for c in range(_NICHUNK):
            nxt = fire(c + 1) if c + 1 < _NICHUNK else None
            buf = c % 2
            out_sl = pl.ds(base + c * _ICHUNK, _ICHUNK)
            cps[0].wait()
            pltpu.sync_copy(ulines_v.at[buf], uout_hbm.at[out_sl])
            cps[1].wait()
            pltpu.sync_copy(mlines_v.at[buf], mout_hbm.at[out_sl])
            cps = nxt

    return k(user_id, movie_id, utab128, mtab128)


def _mlp_body(u_ref, m_ref, uoh_ref, moh_ref, w1u_ref, w1m_ref, b1_ref,
              w2_ref, b2_ref, o_ref):
    uoh = uoh_ref[...]
    moh = moh_ref[...]
    xu = (uoh[:, 0:1] * u_ref[:, :EMBED]
          + uoh[:, 1:2] * u_ref[:, EMBED:2 * EMBED])
    xm = (moh[:, 0:1] * m_ref[:, :EMBED]
          + moh[:, 1:2] * m_ref[:, EMBED:2 * EMBED])
    x = (jnp.dot(xu, w1u_ref[...],
                 preferred_element_type=jnp.float32)
         + jnp.dot(xm, w1m_ref[...],
                   preferred_element_type=jnp.float32)
         + b1_ref[...])
    h = jnp.maximum(x, 0.0)
    o_ref[...] = (jnp.dot(h, w2_ref[...], preferred_element_type=jnp.float32)
                  + b2_ref[...])


def _tc_mlp(u128, m128, uoh, moh, W1u, W1m, b1, W2, b2, block_m=2048):
    grid = (BATCH // block_m,)
    return pl.pallas_call(
        _mlp_body,
        grid=grid,
        in_specs=[
            pl.BlockSpec((block_m, 128), lambda i: (i, 0)),
            pl.BlockSpec((block_m, 128), lambda i: (i, 0)),
            pl.BlockSpec((block_m, 2), lambda i: (i, 0)),
            pl.BlockSpec((block_m, 2), lambda i: (i, 0)),
            pl.BlockSpec((EMBED, HIDDEN), lambda i: (0, 0)),
            pl.BlockSpec((EMBED, HIDDEN), lambda i: (0, 0)),
            pl.BlockSpec((1, HIDDEN), lambda i: (0, 0)),
            pl.BlockSpec((HIDDEN, 1), lambda i: (0, 0)),
            pl.BlockSpec((1, 1), lambda i: (0, 0)),
        ],
        out_specs=pl.BlockSpec((block_m, 1), lambda i: (i, 0)),
        out_shape=jax.ShapeDtypeStruct((BATCH, 1), jnp.float32),
    )(u128, m128, uoh, moh, W1u, W1m, b1, W2, b2)


def kernel(user_id, movie_title, user_table, movie_table, W1, b1, W2, b2):
    uid = user_id.astype(jnp.int32)
    mid = movie_title.astype(jnp.int32)
    utab128 = jnp.pad(user_table.reshape(-1, 2 * EMBED),
                      ((0, 0), (0, 128 - 2 * EMBED)))
    mtab128 = jnp.pad(movie_table.reshape(-1, 2 * EMBED),
                      ((0, 0), (0, 128 - 2 * EMBED)))
    ug = lax.shift_right_logical(uid, 1)
    mg = lax.shift_right_logical(mid, 1)
    u128, m128 = _sc_gather(ug, mg, utab128, mtab128)
    r2 = jnp.arange(2, dtype=jnp.int32)
    uoh = (jnp.bitwise_and(uid, 1)[:, None] == r2).astype(jnp.float32)
    moh = (jnp.bitwise_and(mid, 1)[:, None] == r2).astype(jnp.float32)
    W1u = W1[:EMBED]
    W1m = W1[EMBED:]
    return _tc_mlp(u128, m128, uoh, moh, W1u, W1m,
                   b1.reshape(1, HIDDEN), W2, b2.reshape(1, 1))
